# initial kernel scaffold (unmeasured)
import jax
import jax.numpy as jnp
from jax import lax
from jax.experimental import pallas as pl
from jax.experimental.pallas import tpu as pltpu

N_DEV = 16
M_BLK = 512
K_BLK = 512
N_OUT = 4096


def kernel(x, w_mat):
    m_full, k_blk = x.shape
    _, n_out = w_mat.shape

    def body(x_ref, w_ref, o_ref, recv_buf, w_vmem, send_sems, recv_sems,
             w_sems):
        my = lax.axis_index("i")

        barrier = pltpu.get_barrier_semaphore()
        for j in range(N_DEV):
            @pl.when(my != j)
            def _():
                pl.semaphore_signal(
                    barrier, inc=1,
                    device_id=(j,), device_id_type=pl.DeviceIdType.MESH,
                )
        pl.semaphore_wait(barrier, N_DEV - 1)

        for j in range(N_DEV):
            @pl.when(my != j)
            def _():
                pltpu.make_async_remote_copy(
                    src_ref=x_ref.at[pl.ds(j * M_BLK, M_BLK), :],
                    dst_ref=recv_buf.at[my],
                    send_sem=send_sems.at[j],
                    recv_sem=recv_sems.at[my],
                    device_id=(j,),
                    device_id_type=pl.DeviceIdType.MESH,
                ).start()

        def w_copy(k, slot):
            return pltpu.make_async_copy(
                w_ref.at[pl.ds(k * K_BLK, K_BLK), :],
                w_vmem.at[slot],
                w_sems.at[slot],
            )

        w_copy(0, 0).start()
        for k in range(N_DEV):
            if k + 1 < N_DEV:
                w_copy(k + 1, (k + 1) % 2).start()
            w_copy(k, k % 2).wait()

            def accum(val):
                if k == 0:
                    o_ref[...] = val
                else:
                    o_ref[...] = o_ref[...] + val

            @pl.when(my == k)
            def _():
                accum(jnp.dot(x_ref[pl.ds(k * M_BLK, M_BLK), :],
                              w_vmem[k % 2],
                              preferred_element_type=jnp.float32))

            @pl.when(my != k)
            def _():
                pltpu.make_async_remote_copy(
                    src_ref=x_ref.at[pl.ds(0, M_BLK), :],
                    dst_ref=recv_buf.at[k],
                    send_sem=send_sems.at[k],
                    recv_sem=recv_sems.at[k],
                    device_id=(0,),
                    device_id_type=pl.DeviceIdType.MESH,
                ).wait_recv()
                accum(jnp.dot(recv_buf[k], w_vmem[k % 2],
                              preferred_element_type=jnp.float32))

        y = o_ref[...]
        c = 0.7978845608028654
        o_ref[...] = 0.5 * y * (1.0 + jnp.tanh(c * (y + 0.044715 * y * y * y)))

        for j in range(N_DEV):
            @pl.when(my != j)
            def _():
                pltpu.make_async_remote_copy(
                    src_ref=x_ref.at[pl.ds(j * M_BLK, M_BLK), :],
                    dst_ref=recv_buf.at[my],
                    send_sem=send_sems.at[j],
                    recv_sem=recv_sems.at[my],
                    device_id=(j,),
                    device_id_type=pl.DeviceIdType.MESH,
                ).wait_send()

    return pl.pallas_call(
        body,
        out_shape=jax.ShapeDtypeStruct((M_BLK, n_out), jnp.float32),
        in_specs=[
            pl.BlockSpec(memory_space=pltpu.VMEM),
            pl.BlockSpec(memory_space=pltpu.ANY),
        ],
        out_specs=pl.BlockSpec(memory_space=pltpu.VMEM),
        scratch_shapes=[
            pltpu.VMEM((N_DEV, M_BLK, K_BLK), jnp.float32),
            pltpu.VMEM((2, K_BLK, N_OUT), jnp.float32),
            pltpu.SemaphoreType.DMA((N_DEV,)),
            pltpu.SemaphoreType.DMA((N_DEV,)),
            pltpu.SemaphoreType.DMA((2,)),
        ],
        compiler_params=pltpu.CompilerParams(collective_id=0),
    )(x, w_mat)


# baseline (device time: 215734 ns/iter reference)
import jax
import jax.numpy as jnp
from jax import lax
from jax.experimental import pallas as pl
from jax.experimental.pallas import tpu as pltpu

N_DEV = 16
M_BLK = 512
K_BLK = 512
N_OUT = 4096


def kernel(x, w_mat):
    m_full, k_blk = x.shape
    _, n_out = w_mat.shape

    def body(x_ref, w_ref, o_ref, recv_buf, w_vmem, send_sems, recv_sems,
             w_sems):
        my = lax.axis_index("i")

        barrier = pltpu.get_barrier_semaphore()
        for j in range(N_DEV):
            @pl.when(my != j)
            def _():
                pl.semaphore_signal(
                    barrier, inc=1,
                    device_id=(j,), device_id_type=pl.DeviceIdType.MESH,
                )
        pl.semaphore_wait(barrier, N_DEV - 1)

        for j in range(N_DEV):
            @pl.when(my != j)
            def _():
                pltpu.make_async_remote_copy(
                    src_ref=x_ref.at[pl.ds(j * M_BLK, M_BLK), :],
                    dst_ref=recv_buf.at[my],
                    send_sem=send_sems.at[j],
                    recv_sem=recv_sems.at[my],
                    device_id=(j,),
                    device_id_type=pl.DeviceIdType.MESH,
                ).start()

        def w_copy(k, slot):
            return pltpu.make_async_copy(
                w_ref.at[pl.ds(k * K_BLK, K_BLK), :],
                w_vmem.at[slot],
                w_sems.at[slot],
            )

        w_copy(0, 0).start()
        for k in range(N_DEV):
            if k + 1 < N_DEV:
                w_copy(k + 1, (k + 1) % 2).start()
            w_copy(k, k % 2).wait()

            def accum(val):
                if k == 0:
                    o_ref[...] = val
                else:
                    o_ref[...] = o_ref[...] + val

            @pl.when(my == k)
            def _():
                accum(jnp.dot(x_ref[pl.ds(k * M_BLK, M_BLK), :],
                              w_vmem[k % 2],
                              preferred_element_type=jnp.float32))

            @pl.when(my != k)
            def _():
                pltpu.make_async_remote_copy(
                    src_ref=x_ref.at[pl.ds(0, M_BLK), :],
                    dst_ref=recv_buf.at[k],
                    send_sem=send_sems.at[k],
                    recv_sem=recv_sems.at[k],
                    device_id=(0,),
                    device_id_type=pl.DeviceIdType.MESH,
                ).wait_recv()
                accum(jnp.dot(recv_buf[k], w_vmem[k % 2],
                              preferred_element_type=jnp.float32))

        y = o_ref[...]
        c = 0.7978845608028654
        o_ref[...] = 0.5 * y * (1.0 + jnp.tanh(c * (y + 0.044715 * y * y * y)))

        for j in range(N_DEV):
            @pl.when(my != j)
            def _():
                pltpu.make_async_remote_copy(
                    src_ref=x_ref.at[pl.ds(j * M_BLK, M_BLK), :],
                    dst_ref=recv_buf.at[my],
                    send_sem=send_sems.at[j],
                    recv_sem=recv_sems.at[my],
                    device_id=(j,),
                    device_id_type=pl.DeviceIdType.MESH,
                ).wait_send()

    return pl.pallas_call(
        body,
        out_shape=jax.ShapeDtypeStruct((M_BLK, n_out), jnp.float32),
        in_specs=[
            pl.BlockSpec(memory_space=pltpu.VMEM),
            pl.BlockSpec(memory_space=pltpu.MemorySpace.HBM),
        ],
        out_specs=pl.BlockSpec(memory_space=pltpu.VMEM),
        scratch_shapes=[
            pltpu.VMEM((N_DEV, M_BLK, K_BLK), jnp.float32),
            pltpu.VMEM((2, K_BLK, N_OUT), jnp.float32),
            pltpu.SemaphoreType.DMA((N_DEV,)),
            pltpu.SemaphoreType.DMA((N_DEV,)),
            pltpu.SemaphoreType.DMA((2,)),
        ],
        compiler_params=pltpu.CompilerParams(
            collective_id=0,
            vmem_limit_bytes=100 * 1024 * 1024,
        ),
    )(x, w_mat)
